# Initial kernel scaffold; baseline (speedup 1.0000x reference)
import jax
import jax.numpy as jnp
from jax.experimental import pallas as pl


def kernel(x, edge_index, batch, W1, b1, W2, b2):
    raise NotImplementedError("write your pallas kernel here")


# speedup vs baseline: 29.4562x; 29.4562x over previous
import functools

import jax
import jax.numpy as jnp
from jax import lax
from jax.experimental import pallas as pl
from jax.experimental.pallas import tpu as pltpu
from jax.experimental.pallas import tpu_sc as plsc

N = 10000
E = 320000
D = 128
DH = D // 2
G = 64

NC = 2
NS = 16
CHUNK = 128
CPT = 160
CPT_DEG = CPT // NC
E_PAD = NS * CPT * CHUNK
NBUF = 4
PAD_ROWS = 624
ACC_ROWS = N + PAD_ROWS
DEG_ROWS = 10624
ACC_PT = ACC_ROWS // NS
DEG_PT = DEG_ROWS // NS
OUT_TILES = 10
OUT_PT = N // OUT_TILES

ROWS_BLK = 2000
N_BLKS = N // ROWS_BLK

_mesh = plsc.VectorSubcoreMesh(core_axis_name="c", subcore_axis_name="s")
_sc_params = pltpu.CompilerParams(use_tc_tiling_on_sc=False)


@functools.partial(
    pl.kernel,
    out_type=jax.ShapeDtypeStruct((NC, N, DH), jnp.float32),
    mesh=_mesh,
    scratch_types=[
        pltpu.VMEM((CPT_DEG, CHUNK), jnp.int32),
        pltpu.VMEM((CHUNK, DH), jnp.float32),
        pltpu.VMEM_SHARED((DEG_ROWS, DH), jnp.float32),
    ],
    compiler_params=_sc_params,
)
def _sc_deg(dst_hbm, zeros_hbm, ones_hbm, out_hbm, dst_v, ones_v, deg_sp):
    c = lax.axis_index("c")
    s = lax.axis_index("s")
    pltpu.sync_copy(zeros_hbm.at[pl.ds(s * DEG_PT, DEG_PT)],
                    deg_sp.at[pl.ds(s * DEG_PT, DEG_PT)])
    pltpu.sync_copy(dst_hbm.at[s].at[pl.ds(c * CPT_DEG, CPT_DEG)], dst_v)
    pltpu.sync_copy(ones_hbm, ones_v)
    plsc.subcore_barrier()

    def body(j, carry):
        pltpu.sync_copy(ones_v, deg_sp.at[dst_v.at[j]], add=True)
        return carry

    lax.fori_loop(0, CPT_DEG, body, 0)
    plsc.subcore_barrier()

    @pl.when(s < OUT_TILES)
    def _():
        pltpu.sync_copy(deg_sp.at[pl.ds(s * OUT_PT, OUT_PT)],
                        out_hbm.at[c].at[pl.ds(s * OUT_PT, OUT_PT)])


@functools.partial(
    pl.kernel,
    out_type=jax.ShapeDtypeStruct((NC, N, DH), jnp.float32),
    mesh=_mesh,
    scratch_types=[
        pltpu.VMEM((CPT, CHUNK), jnp.int32),
        pltpu.VMEM((CPT, CHUNK), jnp.int32),
        pltpu.VMEM((NBUF, CHUNK, DH), jnp.float32),
        pltpu.VMEM_SHARED((ACC_ROWS, DH), jnp.float32),
    ] + [pltpu.SemaphoreType.DMA] * NBUF,
    compiler_params=_sc_params,
)
def _sc_spmm(u_hbm, src_hbm, dst_hbm, zeros_hbm, out_hbm,
             src_v, dst_v, rows_v, acc_sp, *gsems):
    c = lax.axis_index("c")
    s = lax.axis_index("s")
    u_c = u_hbm.at[c]
    pltpu.sync_copy(zeros_hbm.at[pl.ds(s * ACC_PT, ACC_PT)],
                    acc_sp.at[pl.ds(s * ACC_PT, ACC_PT)])
    pltpu.sync_copy(src_hbm.at[s], src_v)
    pltpu.sync_copy(dst_hbm.at[s], dst_v)
    plsc.subcore_barrier()

    for b in range(NBUF):
        pltpu.async_copy(u_c.at[src_v.at[b]], rows_v.at[b], gsems[b])

    def body(g, carry):
        j0 = g * NBUF
        for b in range(NBUF):
            j = j0 + b
            pltpu.make_async_copy(u_c.at[pl.ds(0, CHUNK)], rows_v.at[b],
                                  gsems[b]).wait()
            pltpu.sync_copy(rows_v.at[b], acc_sp.at[dst_v.at[j]], add=True)
            nj = j + NBUF

            @pl.when(nj < CPT)
            def _():
                pltpu.async_copy(u_c.at[src_v.at[nj]], rows_v.at[b],
                                 gsems[b])
        return carry

    lax.fori_loop(0, CPT // NBUF, body, 0)
    plsc.subcore_barrier()

    @pl.when(s < OUT_TILES)
    def _():
        pltpu.sync_copy(acc_sp.at[pl.ds(s * OUT_PT, OUT_PT)],
                        out_hbm.at[c].at[pl.ds(s * OUT_PT, OUT_PT)])


def _dinv_of(deg_blk):
    deg = deg_blk[0, :, :1] + deg_blk[1, :, :1] + 1.0
    return 1.0 / jnp.sqrt(deg)


def _tc_u1_body(x_ref, w1a_ref, w1b_ref, deg_ref, o_ref):
    dinv = _dinv_of(deg_ref[...])
    x = x_ref[...]
    o_ref[0] = jnp.dot(x, w1a_ref[...],
                       preferred_element_type=jnp.float32) * dinv
    o_ref[1] = jnp.dot(x, w1b_ref[...],
                       preferred_element_type=jnp.float32) * dinv


def _tc_u2_body(acc_ref, u1_ref, deg_ref, w2a_ref, w2b_ref, b1_ref, o_ref):
    dinv = _dinv_of(deg_ref[...])
    a = acc_ref[...]
    u1 = u1_ref[...]
    h1 = jnp.concatenate([a[0] + u1[0], a[1] + u1[1]], axis=1)
    z1 = jax.nn.relu(h1 * dinv + b1_ref[...])
    o_ref[0] = jnp.dot(z1, w2a_ref[...],
                       preferred_element_type=jnp.float32) * dinv
    o_ref[1] = jnp.dot(z1, w2b_ref[...],
                       preferred_element_type=jnp.float32) * dinv


def _tc_pool_body(acc_ref, u2_ref, deg_ref, b2_ref, batch_ref, o_ref,
                  sums_ref, cnts_ref):
    i = pl.program_id(0)

    @pl.when(i == 0)
    def _():
        sums_ref[...] = jnp.zeros_like(sums_ref)
        cnts_ref[...] = jnp.zeros_like(cnts_ref)

    dinv = _dinv_of(deg_ref[...])
    a = acc_ref[...]
    u2 = u2_ref[...]
    h2 = jnp.concatenate([a[0] + u2[0], a[1] + u2[1]], axis=1)
    z2 = h2 * dinv + b2_ref[...]
    gids = lax.broadcasted_iota(jnp.int32, (ROWS_BLK, G), 1)
    onehot = (batch_ref[...] == gids).astype(jnp.float32)
    dn = (((0,), (0,)), ((), ()))
    sums_ref[...] += lax.dot_general(onehot, z2, dn,
                                     preferred_element_type=jnp.float32)
    cnts_ref[...] += lax.dot_general(onehot, jnp.ones_like(z2), dn,
                                     preferred_element_type=jnp.float32)

    @pl.when(i == N_BLKS - 1)
    def _():
        o_ref[...] = sums_ref[...] / jnp.maximum(cnts_ref[...], 1.0)


_row_spec = pl.BlockSpec((ROWS_BLK, D), lambda i: (i, 0))
_deg_spec = pl.BlockSpec((NC, ROWS_BLK, DH), lambda i: (0, i, 0))
_u_spec = pl.BlockSpec((NC, ROWS_BLK, DH), lambda i: (0, i, 0))
_wh_spec = pl.BlockSpec((D, DH), lambda i: (0, 0))
_b_spec = pl.BlockSpec((1, D), lambda i: (0, 0))

_tc_u1 = pl.pallas_call(
    _tc_u1_body,
    grid=(N_BLKS,),
    in_specs=[_row_spec, _wh_spec, _wh_spec, _deg_spec],
    out_specs=_u_spec,
    out_shape=jax.ShapeDtypeStruct((NC, N, DH), jnp.float32),
)

_tc_u2 = pl.pallas_call(
    _tc_u2_body,
    grid=(N_BLKS,),
    in_specs=[_u_spec, _u_spec, _deg_spec, _wh_spec, _wh_spec, _b_spec],
    out_specs=_u_spec,
    out_shape=jax.ShapeDtypeStruct((NC, N, DH), jnp.float32),
)

_tc_pool = pl.pallas_call(
    _tc_pool_body,
    grid=(N_BLKS,),
    in_specs=[_u_spec, _u_spec, _deg_spec, _b_spec,
              pl.BlockSpec((ROWS_BLK, 1), lambda i: (i, 0))],
    out_specs=pl.BlockSpec((G, D), lambda i: (0, 0)),
    out_shape=jax.ShapeDtypeStruct((G, D), jnp.float32),
    scratch_shapes=[pltpu.VMEM((G, D), jnp.float32),
                    pltpu.VMEM((G, D), jnp.float32)],
)


def kernel(x, edge_index, batch, W1, b1, W2, b2):
    src = edge_index[0]
    dst = edge_index[1]
    npad = E_PAD - E
    pad_i = jnp.arange(npad, dtype=jnp.int32)
    src_p = jnp.concatenate([src, pad_i % N]).reshape(NS, CPT, CHUNK)
    dst_p = jnp.concatenate([dst, N + (pad_i % PAD_ROWS)]).reshape(
        NS, CPT, CHUNK)

    ones = jnp.ones((CHUNK, DH), jnp.float32)
    zacc = jnp.zeros((ACC_ROWS, DH), jnp.float32)
    w1a, w1b = W1[:, :DH], W1[:, DH:]
    w2a, w2b = W2[:, :DH], W2[:, DH:]
    b1r = b1.reshape(1, D)
    b2r = b2.reshape(1, D)
    batch_c = batch.astype(jnp.int32).reshape(N, 1)

    deg = _sc_deg(dst_p, zacc, ones)
    u1 = _tc_u1(x, w1a, w1b, deg)
    acc1 = _sc_spmm(u1, src_p, dst_p, zacc)
    u2 = _tc_u2(acc1, u1, deg, w2a, w2b, b1r)
    acc2 = _sc_spmm(u2, src_p, dst_p, zacc)
    return _tc_pool(acc2, u2, deg, b2r, batch_c)


# speedup vs baseline: 31.6745x; 1.0753x over previous
import functools

import jax
import jax.numpy as jnp
from jax import lax
from jax.experimental import pallas as pl
from jax.experimental.pallas import tpu as pltpu
from jax.experimental.pallas import tpu_sc as plsc

N = 10000
E = 320000
D = 128
DH = D // 2
DW = 16
G = 64

NC = 2
NS = 16
CHUNK = 128
CPT = 160
CPT_DEG = CPT // NC
E_PAD = NS * CPT * CHUNK
NBUF = 4
PAD_ROWS = 240
ACC_ROWS = N + PAD_ROWS
DEG_ROWS = 10240
ACC_PT = ACC_ROWS // NS
DEG_PT = DEG_ROWS // NS
OUT_TILES = 10
OUT_PT = N // OUT_TILES

ROWS_BLK = 2000
N_BLKS = N // ROWS_BLK

_mesh = plsc.VectorSubcoreMesh(core_axis_name="c", subcore_axis_name="s")
_sc_params = pltpu.CompilerParams(use_tc_tiling_on_sc=False)


@functools.partial(
    pl.kernel,
    out_type=jax.ShapeDtypeStruct((NC, N, DW), jnp.float32),
    mesh=_mesh,
    scratch_types=[
        pltpu.VMEM((CPT_DEG, CHUNK), jnp.int32),
        pltpu.VMEM((CHUNK, DW), jnp.float32),
        pltpu.VMEM_SHARED((DEG_ROWS, DW), jnp.float32),
    ],
    compiler_params=_sc_params,
)
def _sc_deg(dst_hbm, zeros_hbm, ones_hbm, out_hbm, dst_v, ones_v, deg_sp):
    c = lax.axis_index("c")
    s = lax.axis_index("s")
    pltpu.sync_copy(zeros_hbm.at[pl.ds(s * DEG_PT, DEG_PT)],
                    deg_sp.at[pl.ds(s * DEG_PT, DEG_PT)])
    pltpu.sync_copy(dst_hbm.at[s].at[pl.ds(c * CPT_DEG, CPT_DEG)], dst_v)
    pltpu.sync_copy(ones_hbm, ones_v)
    plsc.subcore_barrier()

    def body(j, carry):
        pltpu.sync_copy(ones_v, deg_sp.at[dst_v.at[j]], add=True)
        return carry

    lax.fori_loop(0, CPT_DEG, body, 0)
    plsc.subcore_barrier()

    @pl.when(s < OUT_TILES)
    def _():
        pltpu.sync_copy(deg_sp.at[pl.ds(s * OUT_PT, OUT_PT)],
                        out_hbm.at[c].at[pl.ds(s * OUT_PT, OUT_PT)])


@functools.partial(
    pl.kernel,
    out_type=jax.ShapeDtypeStruct((NC, N, DH), jnp.float32),
    mesh=_mesh,
    scratch_types=[
        pltpu.VMEM((CPT, CHUNK), jnp.int32),
        pltpu.VMEM((CPT, CHUNK), jnp.int32),
        pltpu.VMEM((NBUF, CHUNK, DH), jnp.float32),
        pltpu.VMEM_SHARED((ACC_ROWS, DH), jnp.float32),
    ] + [pltpu.SemaphoreType.DMA] * NBUF,
    compiler_params=_sc_params,
)
def _sc_spmm(u_hbm, src_hbm, dst_hbm, zeros_hbm, out_hbm,
             src_v, dst_v, rows_v, acc_sp, *gsems):
    c = lax.axis_index("c")
    s = lax.axis_index("s")
    u_c = u_hbm.at[c]
    pltpu.sync_copy(zeros_hbm.at[pl.ds(s * ACC_PT, ACC_PT)],
                    acc_sp.at[pl.ds(s * ACC_PT, ACC_PT)])
    pltpu.sync_copy(src_hbm.at[s], src_v)
    pltpu.sync_copy(dst_hbm.at[s], dst_v)
    plsc.subcore_barrier()

    for b in range(NBUF):
        pltpu.async_copy(u_c.at[src_v.at[b]], rows_v.at[b], gsems[b])

    def body(g, carry):
        j0 = g * NBUF
        for b in range(NBUF):
            j = j0 + b
            pltpu.make_async_copy(u_c.at[pl.ds(0, CHUNK)], rows_v.at[b],
                                  gsems[b]).wait()
            pltpu.sync_copy(rows_v.at[b], acc_sp.at[dst_v.at[j]], add=True)
            nj = j + NBUF

            @pl.when(nj < CPT)
            def _():
                pltpu.async_copy(u_c.at[src_v.at[nj]], rows_v.at[b],
                                 gsems[b])
        return carry

    lax.fori_loop(0, CPT // NBUF, body, 0)
    plsc.subcore_barrier()

    @pl.when(s < OUT_TILES)
    def _():
        pltpu.sync_copy(acc_sp.at[pl.ds(s * OUT_PT, OUT_PT)],
                        out_hbm.at[c].at[pl.ds(s * OUT_PT, OUT_PT)])


def _dinv_of(deg_blk):
    deg = deg_blk[0, :, :1] + deg_blk[1, :, :1] + 1.0
    return 1.0 / jnp.sqrt(deg)


def _tc_u1_body(x_ref, w1a_ref, w1b_ref, deg_ref, o_ref):
    dinv = _dinv_of(deg_ref[...])
    x = x_ref[...]
    o_ref[0] = jnp.dot(x, w1a_ref[...],
                       preferred_element_type=jnp.float32) * dinv
    o_ref[1] = jnp.dot(x, w1b_ref[...],
                       preferred_element_type=jnp.float32) * dinv


def _tc_u2_body(acc_ref, u1_ref, deg_ref, w2a_ref, w2b_ref, b1_ref, o_ref):
    dinv = _dinv_of(deg_ref[...])
    a = acc_ref[...]
    u1 = u1_ref[...]
    h1 = jnp.concatenate([a[0] + u1[0], a[1] + u1[1]], axis=1)
    z1 = jax.nn.relu(h1 * dinv + b1_ref[...])
    o_ref[0] = jnp.dot(z1, w2a_ref[...],
                       preferred_element_type=jnp.float32) * dinv
    o_ref[1] = jnp.dot(z1, w2b_ref[...],
                       preferred_element_type=jnp.float32) * dinv


def _tc_pool_body(acc_ref, u2_ref, deg_ref, b2_ref, batch_ref, o_ref,
                  sums_ref, cnts_ref):
    i = pl.program_id(0)

    @pl.when(i == 0)
    def _():
        sums_ref[...] = jnp.zeros_like(sums_ref)
        cnts_ref[...] = jnp.zeros_like(cnts_ref)

    dinv = _dinv_of(deg_ref[...])
    a = acc_ref[...]
    u2 = u2_ref[...]
    h2 = jnp.concatenate([a[0] + u2[0], a[1] + u2[1]], axis=1)
    z2 = h2 * dinv + b2_ref[...]
    gids = lax.broadcasted_iota(jnp.int32, (ROWS_BLK, G), 1)
    onehot = (batch_ref[...] == gids).astype(jnp.float32)
    dn = (((0,), (0,)), ((), ()))
    sums_ref[...] += lax.dot_general(onehot, z2, dn,
                                     preferred_element_type=jnp.float32)
    cnts_ref[...] += lax.dot_general(onehot, jnp.ones_like(z2), dn,
                                     preferred_element_type=jnp.float32)

    @pl.when(i == N_BLKS - 1)
    def _():
        o_ref[...] = sums_ref[...] / jnp.maximum(cnts_ref[...], 1.0)


_row_spec = pl.BlockSpec((ROWS_BLK, D), lambda i: (i, 0))
_deg_spec = pl.BlockSpec((NC, ROWS_BLK, DW), lambda i: (0, i, 0))
_u_spec = pl.BlockSpec((NC, ROWS_BLK, DH), lambda i: (0, i, 0))
_wh_spec = pl.BlockSpec((D, DH), lambda i: (0, 0))
_b_spec = pl.BlockSpec((1, D), lambda i: (0, 0))

_tc_u1 = pl.pallas_call(
    _tc_u1_body,
    grid=(N_BLKS,),
    in_specs=[_row_spec, _wh_spec, _wh_spec, _deg_spec],
    out_specs=_u_spec,
    out_shape=jax.ShapeDtypeStruct((NC, N, DH), jnp.float32),
)

_tc_u2 = pl.pallas_call(
    _tc_u2_body,
    grid=(N_BLKS,),
    in_specs=[_u_spec, _u_spec, _deg_spec, _wh_spec, _wh_spec, _b_spec],
    out_specs=_u_spec,
    out_shape=jax.ShapeDtypeStruct((NC, N, DH), jnp.float32),
)

_tc_pool = pl.pallas_call(
    _tc_pool_body,
    grid=(N_BLKS,),
    in_specs=[_u_spec, _u_spec, _deg_spec, _b_spec,
              pl.BlockSpec((ROWS_BLK, 1), lambda i: (i, 0))],
    out_specs=pl.BlockSpec((G, D), lambda i: (0, 0)),
    out_shape=jax.ShapeDtypeStruct((G, D), jnp.float32),
    scratch_shapes=[pltpu.VMEM((G, D), jnp.float32),
                    pltpu.VMEM((G, D), jnp.float32)],
)


def kernel(x, edge_index, batch, W1, b1, W2, b2):
    src = edge_index[0]
    dst = edge_index[1]
    npad = E_PAD - E
    pad_i = jnp.arange(npad, dtype=jnp.int32)
    src_p = jnp.concatenate([src, pad_i % N]).reshape(NS, CPT, CHUNK)
    dst_p = jnp.concatenate([dst, N + (pad_i % PAD_ROWS)]).reshape(
        NS, CPT, CHUNK)

    zdeg = jnp.zeros((DEG_ROWS, DW), jnp.float32)
    ones = jnp.ones((CHUNK, DW), jnp.float32)
    zacc = jnp.zeros((ACC_ROWS, DH), jnp.float32)
    w1a, w1b = W1[:, :DH], W1[:, DH:]
    w2a, w2b = W2[:, :DH], W2[:, DH:]
    b1r = b1.reshape(1, D)
    b2r = b2.reshape(1, D)
    batch_c = batch.astype(jnp.int32).reshape(N, 1)

    deg = _sc_deg(dst_p, zdeg, ones)
    u1 = _tc_u1(x, w1a, w1b, deg)
    acc1 = _sc_spmm(u1, src_p, dst_p, zacc)
    u2 = _tc_u2(acc1, u1, deg, w2a, w2b, b1r)
    acc2 = _sc_spmm(u2, src_p, dst_p, zacc)
    return _tc_pool(acc2, u2, deg, b2r, batch_c)
